# baseline (speedup 1.0000x reference)
import functools

import jax
import jax.numpy as jnp
from jax import lax
from jax.experimental import pallas as pl
from jax.experimental.pallas import tpu as pltpu
from jax.experimental.pallas import tpu_sc as plsc

NUM_CORES = 2
NUM_SUBCORES = 16
LANES = 16
NW = NUM_CORES * NUM_SUBCORES

B = 16384
D = 128
BPW = B // NW
CHUNK = 128
NCHUNK = BPW // CHUNK
GROUPS = CHUNK // LANES
UNROLL = 8

_mesh = plsc.VectorSubcoreMesh(core_axis_name="c", subcore_axis_name="s")


@functools.partial(
    pl.kernel,
    out_type=jax.ShapeDtypeStruct((B,), jnp.float32),
    mesh=_mesh,
    compiler_params=pltpu.CompilerParams(needs_layout_passes=False),
    scratch_types=[
        pltpu.VMEM((NCHUNK, CHUNK), jnp.int32),
        pltpu.VMEM((NCHUNK, CHUNK), jnp.int32),
        pltpu.VMEM((CHUNK, D), jnp.float32),
        pltpu.VMEM((CHUNK, D), jnp.float32),
        pltpu.VMEM((NCHUNK, CHUNK), jnp.float32),
        pltpu.VMEM((NCHUNK, CHUNK), jnp.float32),
        pltpu.VMEM((BPW,), jnp.float32),
        pltpu.VMEM((LANES,), jnp.float32),
        pltpu.SemaphoreType.DMA,
        pltpu.SemaphoreType.DMA,
    ],
)
def _mf_sc(user_hbm, item_hbm, utab_hbm, itab_hbm, ubias_hbm, ibias_hbm,
           off_hbm, out_hbm,
           uidx_v, iidx_v, urows_v, irows_v, ubias_v, ibias_v, out_v,
           off_v, sem, bsem):
    wid = lax.axis_index("s") * NUM_CORES + lax.axis_index("c")
    base = wid * BPW

    pltpu.sync_copy(off_hbm, off_v)
    for c in range(NCHUNK):
        pltpu.sync_copy(user_hbm.at[pl.ds(base + c * CHUNK, CHUNK)],
                        uidx_v.at[c])
        pltpu.sync_copy(item_hbm.at[pl.ds(base + c * CHUNK, CHUNK)],
                        iidx_v.at[c])

    bias_copies = []
    for c in range(NCHUNK):
        bias_copies.append(
            pltpu.async_copy(ubias_hbm.at[uidx_v.at[c]], ubias_v.at[c], bsem))
        bias_copies.append(
            pltpu.async_copy(ibias_hbm.at[iidx_v.at[c]], ibias_v.at[c], bsem))

    off = off_v[...]
    lane_iota = lax.iota(jnp.int32, LANES)

    for c in range(NCHUNK):
        ucp = pltpu.async_copy(utab_hbm.at[uidx_v.at[c]], urows_v, sem)
        icp = pltpu.async_copy(itab_hbm.at[iidx_v.at[c]], irows_v, sem)
        ucp.wait()
        icp.wait()
        for g in range(GROUPS):
            rows = lane_iota + (g * LANES)

            def dbody(j, acc, rows=rows):
                d0 = j * UNROLL
                for k in range(UNROLL):
                    col = jnp.full((LANES,), d0 + k, dtype=jnp.int32)
                    u = plsc.load_gather(urows_v, [rows, col])
                    v = plsc.load_gather(irows_v, [rows, col])
                    acc = acc + u * v
                return acc

            acc = lax.fori_loop(0, D // UNROLL, dbody,
                                jnp.zeros((LANES,), jnp.float32))
            out_v[pl.ds(c * CHUNK + g * LANES, LANES)] = acc

    for cp in bias_copies:
        cp.wait()

    for c in range(NCHUNK):
        for g in range(GROUPS):
            s = pl.ds(g * LANES, LANES)
            x = (out_v[pl.ds(c * CHUNK + g * LANES, LANES)]
                 + ubias_v.at[c][s] + ibias_v.at[c][s] + off)
            y = 5.5 / (1.0 + jnp.exp(-x))
            out_v[pl.ds(c * CHUNK + g * LANES, LANES)] = y

    pltpu.sync_copy(out_v, out_hbm.at[pl.ds(base, BPW)])


@jax.jit
def kernel(user, item, user_emb_table, item_emb_table, user_bias, item_bias,
           offset):
    user = user.astype(jnp.int32)
    item = item.astype(jnp.int32)
    off = jnp.full((LANES,), offset, dtype=jnp.float32)
    return _mf_sc(user, item, user_emb_table, item_emb_table,
                  user_bias, item_bias, off)
